# baseline (speedup 1.0000x reference)
import functools

import jax
import jax.numpy as jnp
from jax import lax
from jax.experimental import pallas as pl
from jax.experimental.pallas import tpu as pltpu
from jax.experimental.pallas import tpu_sc as plsc

_NBUF = 8
_CHO = 64
_CHG = 128


def _schedules(tgt, ctx, b_per_w):
    B = tgt.shape[0]
    idx = jnp.stack([tgt, ctx])
    pos = jnp.broadcast_to(jnp.arange(B, dtype=jnp.int32), (2, B))
    si, sp = lax.sort_key_val(idx, pos)
    slab = lax.shift_right_logical(si, 7)
    col = jnp.bitwise_and(si, 127)
    first = jnp.concatenate(
        [jnp.ones((2, 1), jnp.int32),
         (slab[:, 1:] != slab[:, :-1]).astype(jnp.int32)], axis=1)
    wfirst = jnp.where(pos % b_per_w == 0, 1, first)
    cw = jnp.cumsum(wfirst, axis=1).astype(jnp.int32)
    wstart = (pos // b_per_w) * b_per_w
    rank = cw - jnp.take_along_axis(cw, wstart, axis=1)
    wid = pos // b_per_w
    nd = b_per_w + _NBUF
    nw = B // b_per_w
    t2 = jnp.arange(2, dtype=jnp.int32)[:, None]
    dw = jnp.zeros((2, nw, nd), jnp.int32)
    rr = jnp.where(wfirst == 1, rank, nd - 1)
    dw = dw.at[t2, wid, rr].set(slab, mode="drop")
    flat = dw[:, :, : nd - 1].reshape(2, -1)
    ahead = jnp.take_along_axis(flat, wid * (nd - 1) + rank + (_NBUF - 1),
                                axis=1)
    pf = jnp.where(wfirst == 1, ahead, -1)
    slot = jnp.bitwise_and(rank, _NBUF - 1)
    islot = jnp.bitwise_and(rank + _NBUF - 1, _NBUF - 1)
    pcols = dw[:, :, :16].reshape(2, -1)
    q = jnp.zeros((2, B), jnp.int32).at[t2, sp].set(pos)
    return col, wfirst, pf, slot, islot, pcols, q


@functools.lru_cache(maxsize=None)
def _build1(B, V, D):
    info = plsc.get_sparse_core_info()
    NC, NS = info.num_cores, info.num_subcores
    NW = NC * NS
    b_per_w = B // NW
    mesh = plsc.VectorSubcoreMesh(core_axis_name="c", subcore_axis_name="s")

    @functools.partial(
        pl.kernel,
        mesh=mesh,
        out_type=(
            jax.ShapeDtypeStruct((B, 128), jnp.float32),
            jax.ShapeDtypeStruct((B, 128), jnp.float32),
        ),
        scratch_types=[
            pltpu.VMEM((b_per_w,), jnp.int32),
            pltpu.VMEM((b_per_w,), jnp.int32),
            pltpu.VMEM((b_per_w,), jnp.int32),
            pltpu.VMEM((b_per_w,), jnp.int32),
            pltpu.VMEM((b_per_w,), jnp.int32),
            pltpu.VMEM((16,), jnp.int32),
            pltpu.VMEM((_NBUF, D, 128), jnp.float32),
            pltpu.VMEM((2, _CHO, 128), jnp.float32),
            pltpu.SemaphoreType.DMA,
            pltpu.SemaphoreType.DMA,
        ],
        compiler_params=pltpu.CompilerParams(needs_layout_passes=False),
    )
    def k(wt0_hbm, wt1_hbm, col_hbm, nc_hbm, pf_hbm, sl_hbm, il_hbm, pc_hbm,
          o0_hbm, o1_hbm,
          col_v, nc_v, pf_v, sl_v, il_v, pc_v, slabs_v, out_v, sg, sw):
        wid = lax.axis_index("s") * NC + lax.axis_index("c")
        base = wid * b_per_w
        lanes = lax.iota(jnp.int32, 16)
        n_cho = b_per_w // _CHO

        for t in range(2):
            wt_hbm = (wt0_hbm, wt1_hbm)[t]
            o_hbm = (o0_hbm, o1_hbm)[t]
            pltpu.sync_copy(col_hbm.at[t, pl.ds(base, b_per_w)], col_v)
            pltpu.sync_copy(nc_hbm.at[t, pl.ds(base, b_per_w)], nc_v)
            pltpu.sync_copy(pf_hbm.at[t, pl.ds(base, b_per_w)], pf_v)
            pltpu.sync_copy(sl_hbm.at[t, pl.ds(base, b_per_w)], sl_v)
            pltpu.sync_copy(il_hbm.at[t, pl.ds(base, b_per_w)], il_v)
            pltpu.sync_copy(pc_hbm.at[t, pl.ds(wid * 16, 16)], pc_v)

            pv = pc_v[pl.ds(0, 16)]
            for j in range(_NBUF - 1):
                pltpu.async_copy(
                    wt_hbm.at[:, pl.ds(pv[j] * 128, 128)], slabs_v.at[j], sg)

            for ch in range(n_cho):
                par = ch % 2
                if ch >= 2:
                    pltpu.make_async_copy(
                        out_v.at[par],
                        o_hbm.at[pl.ds(base + (ch - 2) * _CHO, _CHO)],
                        sw).wait()

                def gbody(g, carry, *, ch=ch, par=par, wt_hbm=wt_hbm):
                    off = ch * _CHO + g * 16
                    cv = col_v[pl.ds(off, 16)]
                    nv = nc_v[pl.ds(off, 16)]
                    fv = pf_v[pl.ds(off, 16)]
                    sv = sl_v[pl.ds(off, 16)]
                    iv = il_v[pl.ds(off, 16)]
                    for j in range(16):
                        @pl.when(fv[j] >= 0)
                        def _():
                            pltpu.async_copy(
                                wt_hbm.at[:, pl.ds(fv[j] * 128, 128)],
                                slabs_v.at[iv[j]], sg)

                        @pl.when(nv[j] == 1)
                        def _():
                            pltpu.make_async_copy(
                                wt_hbm.at[:, pl.ds(0, 128)],
                                slabs_v.at[0], sg).wait()

                        i0 = jnp.broadcast_to(sv[j], (16,))
                        i2 = jnp.broadcast_to(cv[j], (16,))
                        for q in range(D // 16):
                            v = plsc.load_gather(
                                slabs_v, [i0, lanes + q * 16, i2])
                            out_v[par, g * 16 + j, pl.ds(q * 16, 16)] = v
                    return carry

                lax.fori_loop(0, _CHO // 16, gbody, 0)
                pltpu.async_copy(
                    out_v.at[par],
                    o_hbm.at[pl.ds(base + ch * _CHO, _CHO)], sw)

            for j in range(_NBUF - 1):
                pltpu.make_async_copy(
                    wt_hbm.at[:, pl.ds(0, 128)], slabs_v.at[0], sg).wait()
            for ch in (n_cho - 2, n_cho - 1):
                pltpu.make_async_copy(
                    out_v.at[ch % 2],
                    o_hbm.at[pl.ds(base + ch * _CHO, _CHO)], sw).wait()

    return k


@functools.lru_cache(maxsize=None)
def _build2(B):
    info = plsc.get_sparse_core_info()
    NC, NS = info.num_cores, info.num_subcores
    NW = NC * NS
    b_per_w = B // NW
    n_ch = b_per_w // _CHG
    mesh = plsc.VectorSubcoreMesh(core_axis_name="c", subcore_axis_name="s")

    @functools.partial(
        pl.kernel,
        mesh=mesh,
        out_type=(
            jax.ShapeDtypeStruct((B, 128), jnp.float32),
            jax.ShapeDtypeStruct((B, 128), jnp.float32),
        ),
        scratch_types=[
            pltpu.VMEM((b_per_w,), jnp.int32),
            pltpu.VMEM((b_per_w,), jnp.int32),
            pltpu.VMEM((_CHG, 128), jnp.float32),
            pltpu.VMEM((_CHG, 128), jnp.float32),
            pltpu.SemaphoreType.DMA,
            pltpu.SemaphoreType.DMA,
            pltpu.SemaphoreType.DMA,
            pltpu.SemaphoreType.DMA,
        ],
    )
    def k(s1_hbm, s2_hbm, p_hbm, o1_hbm, o2_hbm,
          p1_v, p2_v, r1_v, r2_v, g1s, g2s, w1s, w2s):
        wid = lax.axis_index("s") * NC + lax.axis_index("c")
        base = wid * b_per_w
        pltpu.sync_copy(p_hbm.at[0, pl.ds(base, b_per_w)], p1_v)
        pltpu.sync_copy(p_hbm.at[1, pl.ds(base, b_per_w)], p2_v)

        def cbody(c, carry):
            pltpu.async_copy(
                s1_hbm.at[p1_v.at[pl.ds(c * _CHG, _CHG)]], r1_v, g1s).wait()
            w1 = pltpu.async_copy(
                r1_v, o1_hbm.at[pl.ds(base + c * _CHG, _CHG)], w1s)
            pltpu.async_copy(
                s2_hbm.at[p2_v.at[pl.ds(c * _CHG, _CHG)]], r2_v, g2s).wait()
            w2 = pltpu.async_copy(
                r2_v, o2_hbm.at[pl.ds(base + c * _CHG, _CHG)], w2s)
            w1.wait()
            w2.wait()
            return carry

        lax.fori_loop(0, n_ch, cbody, 0)

    return k


def kernel(target, context, W_word, W_out):
    B = target.shape[0]
    V, D = W_word.shape
    NW = 32
    b_per_w = B // NW

    col, nc, pf, sl, il, pc, q = _schedules(target, context, b_per_w)

    scratch1, scratch2 = _build1(B, V, D)(
        W_word.T, W_out.T, col, nc, pf, sl, il, pc)
    o1, o2 = _build2(B)(scratch1, scratch2, q)
    return (o1[:, :D], o2[:, :D])


# speedup vs baseline: 1.0540x; 1.0540x over previous
import functools

import jax
import jax.numpy as jnp
from jax import lax
from jax.experimental import pallas as pl
from jax.experimental.pallas import tpu as pltpu
from jax.experimental.pallas import tpu_sc as plsc

_NBUF = 8
_CHO = 64
_CHG = 128


def _schedule(idx, b_per_w):
    B = idx.shape[0]
    pos = jnp.arange(B, dtype=jnp.int32)
    si, sp = lax.sort_key_val(idx, pos)
    slab = lax.shift_right_logical(si, 7)
    col = jnp.bitwise_and(si, 127)
    first = jnp.concatenate([jnp.ones((1,), jnp.int32),
                             (slab[1:] != slab[:-1]).astype(jnp.int32)])
    wfirst = jnp.where(pos % b_per_w == 0, 1, first)
    cw = jnp.cumsum(wfirst).astype(jnp.int32)
    wstart = (pos // b_per_w) * b_per_w
    rank = cw - jnp.take(cw, wstart)
    wid = pos // b_per_w
    nd = b_per_w + _NBUF
    dw = jnp.zeros((B // b_per_w, nd), jnp.int32)
    rr = jnp.where(wfirst == 1, rank, nd - 1)
    dw = dw.at[wid, rr].set(slab, mode="drop")
    ahead = jnp.take(dw[:, : b_per_w + _NBUF - 1].reshape(-1),
                     wid * (nd - 1) + rank + (_NBUF - 1))
    pf = jnp.where(wfirst == 1, ahead, -1)
    slot = jnp.bitwise_and(rank, _NBUF - 1)
    islot = jnp.bitwise_and(rank + _NBUF - 1, _NBUF - 1)
    pcols = dw[:, :16].reshape(-1)
    return si, sp, col, wfirst, pf, slot, islot, pcols


@functools.lru_cache(maxsize=None)
def _build1(B, V, D):
    info = plsc.get_sparse_core_info()
    NC, NS = info.num_cores, info.num_subcores
    NW = NC * NS
    b_per_w = B // NW
    mesh = plsc.VectorSubcoreMesh(core_axis_name="c", subcore_axis_name="s")

    @functools.partial(
        pl.kernel,
        mesh=mesh,
        out_type=jax.ShapeDtypeStruct((B, 128), jnp.float32),
        scratch_types=[
            pltpu.VMEM((b_per_w,), jnp.int32),
            pltpu.VMEM((b_per_w,), jnp.int32),
            pltpu.VMEM((b_per_w,), jnp.int32),
            pltpu.VMEM((b_per_w,), jnp.int32),
            pltpu.VMEM((b_per_w,), jnp.int32),
            pltpu.VMEM((16,), jnp.int32),
            pltpu.VMEM((_NBUF, D, 128), jnp.float32),
            pltpu.VMEM((2, _CHO, 128), jnp.float32),
            pltpu.SemaphoreType.DMA,
            pltpu.SemaphoreType.DMA,
        ],
        compiler_params=pltpu.CompilerParams(needs_layout_passes=False),
    )
    def k(wt_hbm, col_hbm, nc_hbm, pf_hbm, sl_hbm, il_hbm, pc_hbm, o_hbm,
          col_v, nc_v, pf_v, sl_v, il_v, pc_v, slabs_v, out_v, sg, sw):
        wid = lax.axis_index("s") * NC + lax.axis_index("c")
        base = wid * b_per_w
        pltpu.sync_copy(col_hbm.at[pl.ds(base, b_per_w)], col_v)
        pltpu.sync_copy(nc_hbm.at[pl.ds(base, b_per_w)], nc_v)
        pltpu.sync_copy(pf_hbm.at[pl.ds(base, b_per_w)], pf_v)
        pltpu.sync_copy(sl_hbm.at[pl.ds(base, b_per_w)], sl_v)
        pltpu.sync_copy(il_hbm.at[pl.ds(base, b_per_w)], il_v)
        pltpu.sync_copy(pc_hbm.at[pl.ds(wid * 16, 16)], pc_v)

        pv = pc_v[pl.ds(0, 16)]
        for j in range(_NBUF - 1):
            pltpu.async_copy(
                wt_hbm.at[:, pl.ds(pv[j] * 128, 128)], slabs_v.at[j], sg)

        lanes = lax.iota(jnp.int32, 16)

        n_cho = b_per_w // _CHO
        for ch in range(n_cho):
            par = ch % 2
            if ch >= 2:
                pltpu.make_async_copy(
                    out_v.at[par],
                    o_hbm.at[pl.ds(base + (ch - 2) * _CHO, _CHO)], sw).wait()

            def gbody(g, carry, *, ch=ch, par=par):
                off = ch * _CHO + g * 16
                cv = col_v[pl.ds(off, 16)]
                nv = nc_v[pl.ds(off, 16)]
                fv = pf_v[pl.ds(off, 16)]
                sv = sl_v[pl.ds(off, 16)]
                iv = il_v[pl.ds(off, 16)]
                for j in range(16):
                    @pl.when(fv[j] >= 0)
                    def _():
                        pltpu.async_copy(
                            wt_hbm.at[:, pl.ds(fv[j] * 128, 128)],
                            slabs_v.at[iv[j]], sg)

                    @pl.when(nv[j] == 1)
                    def _():
                        pltpu.make_async_copy(
                            wt_hbm.at[:, pl.ds(0, 128)],
                            slabs_v.at[0], sg).wait()

                    i0 = jnp.broadcast_to(sv[j], (16,))
                    i2 = jnp.broadcast_to(cv[j], (16,))
                    for q in range(D // 16):
                        v = plsc.load_gather(
                            slabs_v, [i0, lanes + q * 16, i2])
                        out_v[par, g * 16 + j, pl.ds(q * 16, 16)] = v
                return carry

            lax.fori_loop(0, _CHO // 16, gbody, 0)
            pltpu.async_copy(
                out_v.at[par], o_hbm.at[pl.ds(base + ch * _CHO, _CHO)], sw)

        for j in range(_NBUF - 1):
            pltpu.make_async_copy(
                wt_hbm.at[:, pl.ds(0, 128)], slabs_v.at[0], sg).wait()
        for ch in (n_cho - 2, n_cho - 1):
            pltpu.make_async_copy(
                out_v.at[ch % 2],
                o_hbm.at[pl.ds(base + ch * _CHO, _CHO)], sw).wait()

    return k


@functools.lru_cache(maxsize=None)
def _build2(B):
    info = plsc.get_sparse_core_info()
    NC, NS = info.num_cores, info.num_subcores
    NW = NC * NS
    b_per_w = B // NW
    n_ch = b_per_w // _CHG
    mesh = plsc.VectorSubcoreMesh(core_axis_name="c", subcore_axis_name="s")

    @functools.partial(
        pl.kernel,
        mesh=mesh,
        out_type=(
            jax.ShapeDtypeStruct((B, 128), jnp.float32),
            jax.ShapeDtypeStruct((B, 128), jnp.float32),
        ),
        scratch_types=[
            pltpu.VMEM((b_per_w,), jnp.int32),
            pltpu.VMEM((b_per_w,), jnp.int32),
            pltpu.VMEM((_CHG, 128), jnp.float32),
            pltpu.VMEM((_CHG, 128), jnp.float32),
            pltpu.SemaphoreType.DMA,
            pltpu.SemaphoreType.DMA,
            pltpu.SemaphoreType.DMA,
            pltpu.SemaphoreType.DMA,
        ],
    )
    def k(s1_hbm, s2_hbm, p1_hbm, p2_hbm, o1_hbm, o2_hbm,
          p1_v, p2_v, r1_v, r2_v, g1s, g2s, w1s, w2s):
        wid = lax.axis_index("s") * NC + lax.axis_index("c")
        base = wid * b_per_w
        pltpu.sync_copy(p1_hbm.at[pl.ds(base, b_per_w)], p1_v)
        pltpu.sync_copy(p2_hbm.at[pl.ds(base, b_per_w)], p2_v)

        def cbody(c, carry):
            pltpu.async_copy(
                s1_hbm.at[p1_v.at[pl.ds(c * _CHG, _CHG)]], r1_v, g1s).wait()
            w1 = pltpu.async_copy(
                r1_v, o1_hbm.at[pl.ds(base + c * _CHG, _CHG)], w1s)
            pltpu.async_copy(
                s2_hbm.at[p2_v.at[pl.ds(c * _CHG, _CHG)]], r2_v, g2s).wait()
            w2 = pltpu.async_copy(
                r2_v, o2_hbm.at[pl.ds(base + c * _CHG, _CHG)], w2s)
            w1.wait()
            w2.wait()
            return carry

        lax.fori_loop(0, n_ch, cbody, 0)

    return k


def kernel(target, context, W_word, W_out):
    B = target.shape[0]
    V, D = W_word.shape
    NW = 32
    b_per_w = B // NW

    s1, p1, c1, n1, f1, l1, i1, pc1 = _schedule(target, b_per_w)
    s2, p2, c2, n2, f2, l2, i2, pc2 = _schedule(context, b_per_w)
    pos = jnp.arange(B, dtype=jnp.int32)
    q1 = jnp.zeros((B,), jnp.int32).at[p1].set(pos)
    q2 = jnp.zeros((B,), jnp.int32).at[p2].set(pos)

    k1 = _build1(B, V, D)
    scratch1 = k1(W_word.T, c1, n1, f1, l1, i1, pc1)
    scratch2 = k1(W_out.T, c2, n2, f2, l2, i2, pc2)
    o1, o2 = _build2(B)(scratch1, scratch2, q1, q2)
    return (o1[:, :D], o2[:, :D])


# speedup vs baseline: 1.1207x; 1.0632x over previous
import functools

import jax
import jax.numpy as jnp
from jax import lax
from jax.experimental import pallas as pl
from jax.experimental.pallas import tpu as pltpu
from jax.experimental.pallas import tpu_sc as plsc

_NBUF = 8
_CHO = 64
_CHG = 128


def _schedule(idx, b_per_w):
    B = idx.shape[0]
    pos = jnp.arange(B, dtype=jnp.int32)
    si, sp = lax.sort_key_val(idx, pos)
    slab = lax.shift_right_logical(si, 7)
    col = jnp.bitwise_and(si, 127)
    first = jnp.concatenate([jnp.ones((1,), jnp.int32),
                             (slab[1:] != slab[:-1]).astype(jnp.int32)])
    wfirst = jnp.where(pos % b_per_w == 0, 1, first)
    cw = jnp.cumsum(wfirst).astype(jnp.int32)
    wstart = (pos // b_per_w) * b_per_w
    rank = cw - jnp.take(cw, wstart)
    wid = pos // b_per_w
    nd = b_per_w + _NBUF
    dw = jnp.zeros((B // b_per_w, nd), jnp.int32)
    rr = jnp.where(wfirst == 1, rank, nd - 1)
    dw = dw.at[wid, rr].set(slab, mode="drop")
    ahead = jnp.take(dw[:, : b_per_w + _NBUF - 1].reshape(-1),
                     wid * (nd - 1) + rank + (_NBUF - 1))
    pf = jnp.where(wfirst == 1, ahead, -1)
    slot = jnp.bitwise_and(rank, _NBUF - 1)
    islot = jnp.bitwise_and(rank + _NBUF - 1, _NBUF - 1)
    pcols = dw[:, :16].reshape(-1)
    return si, sp, col, wfirst, pf, slot, islot, pcols


@functools.lru_cache(maxsize=None)
def _build1(B, V, D):
    info = plsc.get_sparse_core_info()
    NC, NS = info.num_cores, info.num_subcores
    NW = NC * NS
    b_per_w = B // NW
    mesh = plsc.VectorSubcoreMesh(core_axis_name="c", subcore_axis_name="s")

    @functools.partial(
        pl.kernel,
        mesh=mesh,
        out_type=jax.ShapeDtypeStruct((B, 128), jnp.float32),
        scratch_types=[
            pltpu.VMEM((b_per_w,), jnp.int32),
            pltpu.VMEM((b_per_w,), jnp.int32),
            pltpu.VMEM((b_per_w,), jnp.int32),
            pltpu.VMEM((b_per_w,), jnp.int32),
            pltpu.VMEM((b_per_w,), jnp.int32),
            pltpu.VMEM((16,), jnp.int32),
            pltpu.VMEM((b_per_w // _CHO, _CHO), jnp.int32),
            pltpu.VMEM((_NBUF, D, 128), jnp.float32),
            pltpu.VMEM((2, _CHO, 128), jnp.float32),
            pltpu.SemaphoreType.DMA,
            pltpu.SemaphoreType.DMA,
        ],
        compiler_params=pltpu.CompilerParams(needs_layout_passes=False),
    )
    def k(wt_hbm, col_hbm, nc_hbm, pf_hbm, sl_hbm, il_hbm, pc_hbm, sp_hbm,
          o_hbm,
          col_v, nc_v, pf_v, sl_v, il_v, pc_v, sp_v, slabs_v, out_v, sg, sw):
        wid = lax.axis_index("s") * NC + lax.axis_index("c")
        base = wid * b_per_w
        pltpu.sync_copy(col_hbm.at[pl.ds(base, b_per_w)], col_v)
        pltpu.sync_copy(nc_hbm.at[pl.ds(base, b_per_w)], nc_v)
        pltpu.sync_copy(pf_hbm.at[pl.ds(base, b_per_w)], pf_v)
        pltpu.sync_copy(sl_hbm.at[pl.ds(base, b_per_w)], sl_v)
        pltpu.sync_copy(il_hbm.at[pl.ds(base, b_per_w)], il_v)
        pltpu.sync_copy(pc_hbm.at[pl.ds(wid * 16, 16)], pc_v)
        pltpu.sync_copy(sp_hbm.at[wid], sp_v)

        pv = pc_v[pl.ds(0, 16)]
        for j in range(_NBUF - 1):
            pltpu.async_copy(
                wt_hbm.at[:, pl.ds(pv[j] * 128, 128)], slabs_v.at[j], sg)

        lanes = lax.iota(jnp.int32, 16)

        n_cho = b_per_w // _CHO
        for ch in range(n_cho):
            par = ch % 2
            if ch >= 2:
                pltpu.make_async_copy(
                    out_v.at[par], o_hbm.at[sp_v.at[ch - 2]], sw).wait()

            def gbody(g, carry, *, ch=ch, par=par):
                off = ch * _CHO + g * 16
                cv = col_v[pl.ds(off, 16)]
                nv = nc_v[pl.ds(off, 16)]
                fv = pf_v[pl.ds(off, 16)]
                sv = sl_v[pl.ds(off, 16)]
                iv = il_v[pl.ds(off, 16)]
                for j in range(16):
                    @pl.when(fv[j] >= 0)
                    def _():
                        pltpu.async_copy(
                            wt_hbm.at[:, pl.ds(fv[j] * 128, 128)],
                            slabs_v.at[iv[j]], sg)

                    @pl.when(nv[j] == 1)
                    def _():
                        pltpu.make_async_copy(
                            wt_hbm.at[:, pl.ds(0, 128)],
                            slabs_v.at[0], sg).wait()

                    i0 = jnp.broadcast_to(sv[j], (16,))
                    i2 = jnp.broadcast_to(cv[j], (16,))
                    for q in range(D // 16):
                        v = plsc.load_gather(
                            slabs_v, [i0, lanes + q * 16, i2])
                        out_v[par, g * 16 + j, pl.ds(q * 16, 16)] = v
                return carry

            lax.fori_loop(0, _CHO // 16, gbody, 0)
            pltpu.async_copy(out_v.at[par], o_hbm.at[sp_v.at[ch]], sw)

        for j in range(_NBUF - 1):
            pltpu.make_async_copy(
                wt_hbm.at[:, pl.ds(0, 128)], slabs_v.at[0], sg).wait()
        for ch in (n_cho - 2, n_cho - 1):
            pltpu.make_async_copy(
                out_v.at[ch % 2], o_hbm.at[sp_v.at[ch]], sw).wait()

    return k


def kernel(target, context, W_word, W_out):
    B = target.shape[0]
    V, D = W_word.shape
    NW = 32
    b_per_w = B // NW

    s1, p1, c1, n1, f1, l1, i1, pc1 = _schedule(target, b_per_w)
    s2, p2, c2, n2, f2, l2, i2, pc2 = _schedule(context, b_per_w)
    sp1 = p1.reshape(NW, b_per_w // _CHO, _CHO)
    sp2 = p2.reshape(NW, b_per_w // _CHO, _CHO)

    k1 = _build1(B, V, D)
    o1 = k1(W_word.T, c1, n1, f1, l1, i1, pc1, sp1)
    o2 = k1(W_out.T, c2, n2, f2, l2, i2, pc2, sp2)
    return (o1[:, :D], o2[:, :D])


# speedup vs baseline: 1.1226x; 1.0017x over previous
import functools

import jax
import jax.numpy as jnp
from jax import lax
from jax.experimental import pallas as pl
from jax.experimental.pallas import tpu as pltpu
from jax.experimental.pallas import tpu_sc as plsc

_NBUF = 8
_CHO = 64


def _schedule(idx, b_per_w):
    B = idx.shape[0]
    pos = jnp.arange(B, dtype=jnp.int32)
    si, sp = lax.sort_key_val(idx, pos)
    slab = lax.shift_right_logical(si, 7)
    col = jnp.bitwise_and(si, 127)
    first = jnp.concatenate([jnp.ones((1,), jnp.int32),
                             (slab[1:] != slab[:-1]).astype(jnp.int32)])
    wfirst = jnp.where(pos % b_per_w == 0, 1, first)
    cw = jnp.cumsum(wfirst).astype(jnp.int32)
    wstart = (pos // b_per_w) * b_per_w
    rank = cw - jnp.take(cw, wstart)
    wid = pos // b_per_w
    nd = b_per_w + _NBUF
    dw = jnp.zeros((B // b_per_w, nd), jnp.int32)
    rr = jnp.where(wfirst == 1, rank, nd - 1)
    dw = dw.at[wid, rr].set(slab, mode="drop")
    ahead = jnp.take(dw[:, : b_per_w + _NBUF - 1].reshape(-1),
                     wid * (nd - 1) + rank + (_NBUF - 1))
    pf = jnp.where(wfirst == 1, ahead, -1)
    slot = jnp.bitwise_and(rank, _NBUF - 1)
    islot = jnp.bitwise_and(rank + _NBUF - 1, _NBUF - 1)
    pcols = dw[:, :16].reshape(-1)
    return si, sp, col, wfirst, pf, slot, islot, pcols


@functools.lru_cache(maxsize=None)
def _build1(B, V, D):
    info = plsc.get_sparse_core_info()
    NC, NS = info.num_cores, info.num_subcores
    NW = NC * NS
    b_per_w = B // NW
    mesh = plsc.VectorSubcoreMesh(core_axis_name="c", subcore_axis_name="s")

    @functools.partial(
        pl.kernel,
        mesh=mesh,
        out_type=jax.ShapeDtypeStruct((B, 128), jnp.float32),
        scratch_types=[
            pltpu.VMEM((b_per_w,), jnp.int32),
            pltpu.VMEM((b_per_w,), jnp.int32),
            pltpu.VMEM((b_per_w,), jnp.int32),
            pltpu.VMEM((b_per_w,), jnp.int32),
            pltpu.VMEM((b_per_w,), jnp.int32),
            pltpu.VMEM((16,), jnp.int32),
            pltpu.VMEM((b_per_w // _CHO, _CHO), jnp.int32),
            pltpu.VMEM((_NBUF, D, 128), jnp.float32),
            pltpu.VMEM((2, _CHO, 128), jnp.float32),
            pltpu.SemaphoreType.DMA,
            pltpu.SemaphoreType.DMA,
        ],
        compiler_params=pltpu.CompilerParams(needs_layout_passes=False),
    )
    def k(wt_hbm, col_hbm, nc_hbm, pf_hbm, sl_hbm, il_hbm, pc_hbm, sp_hbm,
          o_hbm,
          col_v, nc_v, pf_v, sl_v, il_v, pc_v, sp_v, slabs_v, out_v, sg, sw):
        wid = lax.axis_index("s") * NC + lax.axis_index("c")
        base = wid * b_per_w
        pltpu.sync_copy(col_hbm.at[pl.ds(base, b_per_w)], col_v)
        pltpu.sync_copy(nc_hbm.at[pl.ds(base, b_per_w)], nc_v)
        pltpu.sync_copy(pf_hbm.at[pl.ds(base, b_per_w)], pf_v)
        pltpu.sync_copy(sl_hbm.at[pl.ds(base, b_per_w)], sl_v)
        pltpu.sync_copy(il_hbm.at[pl.ds(base, b_per_w)], il_v)
        pltpu.sync_copy(pc_hbm.at[pl.ds(wid * 16, 16)], pc_v)
        pltpu.sync_copy(sp_hbm.at[wid], sp_v)

        pv = pc_v[pl.ds(0, 16)]
        for j in range(_NBUF - 1):
            pltpu.async_copy(
                wt_hbm.at[:, pl.ds(pv[j] * 128, 128)], slabs_v.at[j], sg)

        lanes = lax.iota(jnp.int32, 16)

        n_cho = b_per_w // _CHO
        for ch in range(n_cho):
            par = ch % 2
            if ch >= 2:
                pltpu.make_async_copy(
                    out_v.at[par], o_hbm.at[sp_v.at[ch - 2]], sw).wait()

            def gbody(g, carry, *, ch=ch, par=par):
                off = ch * _CHO + g * 16
                cv = col_v[pl.ds(off, 16)]
                nv = nc_v[pl.ds(off, 16)]
                fv = pf_v[pl.ds(off, 16)]
                sv = sl_v[pl.ds(off, 16)]
                iv = il_v[pl.ds(off, 16)]
                for j in range(16):
                    @pl.when(fv[j] >= 0)
                    def _():
                        pltpu.async_copy(
                            wt_hbm.at[:, pl.ds(fv[j] * 128, 128)],
                            slabs_v.at[iv[j]], sg)

                    @pl.when(nv[j] == 1)
                    def _():
                        pltpu.make_async_copy(
                            wt_hbm.at[:, pl.ds(0, 128)],
                            slabs_v.at[0], sg).wait()

                    i0 = jnp.broadcast_to(sv[j], (16,))
                    i2 = jnp.broadcast_to(cv[j], (16,))
                    for q in range(D // 16):
                        v = plsc.load_gather(
                            slabs_v, [i0, lanes + q * 16, i2])
                        out_v[par, g * 16 + j, pl.ds(q * 16, 16)] = v
                return carry

            lax.fori_loop(0, _CHO // 16, gbody, 0)
            pltpu.async_copy(out_v.at[par], o_hbm.at[sp_v.at[ch]], sw)

        for j in range(_NBUF - 1):
            pltpu.make_async_copy(
                wt_hbm.at[:, pl.ds(0, 128)], slabs_v.at[0], sg).wait()
        for ch in (n_cho - 2, n_cho - 1):
            pltpu.make_async_copy(
                out_v.at[ch % 2], o_hbm.at[sp_v.at[ch]], sw).wait()

    return k


def kernel(target, context, W_word, W_out):
    B = target.shape[0]
    V, D = W_word.shape
    NW = 32
    b_per_w = B // NW

    s1, p1, c1, n1, f1, l1, i1, pc1 = _schedule(target, b_per_w)
    s2, p2, c2, n2, f2, l2, i2, pc2 = _schedule(context, b_per_w)
    sp1 = p1.reshape(NW, b_per_w // _CHO, _CHO)
    sp2 = p2.reshape(NW, b_per_w // _CHO, _CHO)

    k1 = _build1(B, V, D)
    o1 = k1(W_word.T, c1, n1, f1, l1, i1, pc1, sp1)
    o2 = k1(W_out.T, c2, n2, f2, l2, i2, pc2, sp2)
    return (o1[:, :D], o2[:, :D])
